# baseline (speedup 1.0000x reference)
import functools

import jax
import jax.numpy as jnp
from jax import lax
from jax.experimental import pallas as pl
from jax.experimental.pallas import tpu as pltpu
from jax.experimental.pallas import tpu_sc as plsc

VOCAB = 1000000
DIM = 64
B = 16384
P = 20
N = 100
PN = P + N
PAD = 128

NC = 2
NS = 16
NW = NC * NS
BPW = B // NW
CHUNK = 64


def _sc_body(idx_hbm, tabs_hbm, scores_hbm, idx_v, rows_v, scores_v,
             gsem0, gsem1):
    wid = lax.axis_index("s") * NC + lax.axis_index("c")
    base = wid * BPW

    pltpu.sync_copy(idx_hbm.at[pl.ds(base, BPW)], idx_v)

    def gather_start(b, slot, sem):
        pltpu.async_copy(tabs_hbm.at[idx_v.at[b]], rows_v.at[slot], sem)

    def gather_wait(slot, sem):
        pltpu.make_async_copy(tabs_hbm.at[pl.ds(0, PAD)], rows_v.at[slot], sem).wait()

    lane = lax.iota(jnp.int32, 16)

    def compute(b, slot):
        buf = rows_v.at[slot]
        r0 = buf.at[0]
        c = [r0[pl.ds(k * 16, 16)] for k in range(DIM // 16)]
        ci = b & (CHUNK - 1)

        for grp in range(PAD // 16):
            acc = jnp.zeros((16,), jnp.float32)
            for jl in range(16):
                j = grp * 16 + jl
                if j >= PN:
                    break
                r = buf.at[j + 1]
                s = r[pl.ds(0, 16)] * c[0]
                for k in range(1, DIM // 16):
                    s = s + r[pl.ds(k * 16, 16)] * c[k]
                acc = jnp.where(lane == jl, jnp.sum(s), acc)
            scores_v[ci, pl.ds(grp * 16, 16)] = acc

        @pl.when(ci == CHUNK - 1)
        def _flush():
            off = pl.multiple_of(base + b - (CHUNK - 1), CHUNK)
            pltpu.sync_copy(scores_v, scores_hbm.at[pl.ds(off, CHUNK)])

    gather_start(0, 0, gsem0)

    def tbody(t, _):
        b0 = 2 * t
        b1 = 2 * t + 1
        gather_start(b1, 1, gsem1)
        gather_wait(0, gsem0)
        compute(b0, 0)

        @pl.when(t < BPW // 2 - 1)
        def _():
            gather_start(b1 + 1, 0, gsem0)

        gather_wait(1, gsem1)
        compute(b1, 1)
        return 0

    lax.fori_loop(0, BPW // 2, tbody, 0)


NCH = VOCAB // PAD
ROUNDS = (NCH // NW) & ~1


def _prep_body(inT_hbm, outT_hbm, tail_hbm, tabs_hbm,
               inb_v, outb_v, ob_v, tailb_v, psem0, psem1, wsem0, wsem1):
    wid = lax.axis_index("s") * NC + lax.axis_index("c")

    def fetch(c, s, sem):
        v0 = pl.multiple_of(c * PAD, PAD)
        pltpu.async_copy(inT_hbm.at[:, pl.ds(v0, PAD)], inb_v.at[s], sem)
        pltpu.async_copy(outT_hbm.at[:, pl.ds(v0, PAD)], outb_v.at[s], sem)

    def fetch_wait(s, sem):
        pltpu.make_async_copy(inT_hbm.at[:, pl.ds(0, PAD)], inb_v.at[s], sem).wait()
        pltpu.make_async_copy(outT_hbm.at[:, pl.ds(0, PAD)], outb_v.at[s], sem).wait()

    iotas = [lax.iota(jnp.int32, 16) + 16 * k for k in range(DIM // 16)]

    def transpose(s):
        src_in = inb_v.at[s]
        src_out = outb_v.at[s]
        ob = ob_v.at[s]

        def vbody(vl, _):
            vv = jnp.full((16,), vl, jnp.int32)
            for k in range(DIM // 16):
                ob[vl, pl.ds(16 * k, 16)] = plsc.load_gather(src_in, [iotas[k], vv])
                ob[vl, pl.ds(DIM + 16 * k, 16)] = plsc.load_gather(src_out, [iotas[k], vv])
            return 0

        lax.fori_loop(0, PAD, vbody, 0, unroll=8)

    def write(c, s, sem):
        off = pl.multiple_of(c * PAD, PAD)
        pltpu.async_copy(ob_v.at[s], tabs_hbm.at[pl.ds(off, PAD)], sem)

    def write_wait(s, sem):
        pltpu.make_async_copy(ob_v.at[s], tabs_hbm.at[pl.ds(0, PAD)], sem).wait()

    fetch(wid, 0, psem0)

    def rbody(rr, _):
        c0 = (2 * rr) * NW + wid
        c1 = c0 + NW
        fetch(c1, 1, psem1)
        fetch_wait(0, psem0)

        @pl.when(rr > 0)
        def _():
            write_wait(0, wsem0)

        transpose(0)
        write(c0, 0, wsem0)

        @pl.when(rr < ROUNDS // 2 - 1)
        def _():
            fetch(c1 + NW, 0, psem0)

        fetch_wait(1, psem1)

        @pl.when(rr > 0)
        def _():
            write_wait(1, wsem1)

        transpose(1)
        write(c1, 1, wsem1)
        return 0

    lax.fori_loop(0, ROUNDS // 2, rbody, 0)
    write_wait(0, wsem0)
    write_wait(1, wsem1)

    c2 = ROUNDS * NW + wid

    @pl.when(c2 < NCH)
    def _():
        fetch(c2, 0, psem0)
        fetch_wait(0, psem0)
        transpose(0)
        write(c2, 0, wsem0)
        write_wait(0, wsem0)

    @pl.when(wid == NW - 1)
    def _():
        pltpu.sync_copy(tail_hbm, tailb_v)
        pltpu.sync_copy(tailb_v, tabs_hbm.at[pl.ds(NCH * PAD, VOCAB - NCH * PAD)])


@jax.jit
def _sc_prep(inT, outT, tail):
    mesh = plsc.VectorSubcoreMesh(core_axis_name="c", subcore_axis_name="s")
    return pl.kernel(
        _prep_body,
        out_type=jax.ShapeDtypeStruct((VOCAB, PAD), jnp.float32),
        mesh=mesh,
        compiler_params=pltpu.CompilerParams(
            needs_layout_passes=False,
            use_tc_tiling_on_sc=True,
        ),
        scratch_types=[
            pltpu.VMEM((2, DIM, PAD), jnp.float32),
            pltpu.VMEM((2, DIM, PAD), jnp.float32),
            pltpu.VMEM((2, PAD, PAD), jnp.float32),
            pltpu.VMEM((VOCAB - NCH * PAD, PAD), jnp.float32),
            pltpu.SemaphoreType.DMA,
            pltpu.SemaphoreType.DMA,
            pltpu.SemaphoreType.DMA,
            pltpu.SemaphoreType.DMA,
        ],
    )(inT, outT, tail)


@jax.jit
def _sc_scores(idx, tabs):
    mesh = plsc.VectorSubcoreMesh(core_axis_name="c", subcore_axis_name="s")
    return pl.kernel(
        _sc_body,
        out_type=jax.ShapeDtypeStruct((B, PAD), jnp.float32),
        mesh=mesh,
        compiler_params=pltpu.CompilerParams(
            needs_layout_passes=False,
            use_tc_tiling_on_sc=False,
        ),
        scratch_types=[
            pltpu.VMEM((BPW, PAD), jnp.int32),
            pltpu.VMEM((2, PAD, DIM), jnp.float32),
            pltpu.VMEM((CHUNK, PAD), jnp.float32),
            pltpu.SemaphoreType.DMA,
            pltpu.SemaphoreType.DMA,
        ],
    )(idx, tabs)


def _tc_body(s_ref, o_ref):
    x = s_ref[...]
    col = lax.broadcasted_iota(jnp.int32, x.shape, 1)
    t = jnp.log1p(jnp.exp(-jnp.abs(x))) - jnp.minimum(x, 0.0)
    o_ref[...] = jnp.sum(jnp.where(col < PN, t, 0.0), axis=1)


@jax.jit
def _tc_reduce(scores):
    blk = 2048
    return pl.pallas_call(
        _tc_body,
        grid=(B // blk,),
        in_specs=[pl.BlockSpec((blk, PAD), lambda i: (i, 0))],
        out_specs=pl.BlockSpec((blk,), lambda i: (i,)),
        out_shape=jax.ShapeDtypeStruct((B,), jnp.float32),
    )(scores)


def kernel(center, pos_words, neg_words, in_table, out_table):
    tail = jnp.concatenate(
        [in_table[NCH * PAD:], out_table[NCH * PAD:]], axis=1)
    tabs = _sc_prep(in_table.T, out_table.T, tail).reshape(2 * VOCAB, DIM)
    idx = jnp.concatenate(
        [2 * center[:, None], 2 * pos_words + 1, 2 * neg_words + 1,
         2 * pos_words[:, : PAD - 1 - PN] + 1], axis=1)
    scores = _sc_scores(idx, tabs)
    return _tc_reduce(scores)


# speedup vs baseline: 3.0846x; 3.0846x over previous
import functools

import jax
import jax.numpy as jnp
from jax import lax
from jax.experimental import pallas as pl
from jax.experimental.pallas import tpu as pltpu
from jax.experimental.pallas import tpu_sc as plsc

VOCAB = 1000000
DIM = 64
B = 16384
P = 20
N = 100
PN = P + N
PAD = 128

NC = 2
NS = 16
NW = NC * NS
BPW = B // NW
CHUNK = 64


def _sc_body(idx_hbm, tabs_hbm, scores_hbm, idx_v, rows_v, scores_v,
             gsem0, gsem1):
    wid = lax.axis_index("s") * NC + lax.axis_index("c")
    base = wid * BPW

    pltpu.sync_copy(idx_hbm.at[pl.ds(base, BPW)], idx_v)

    def gather_start(b, slot, sem):
        pltpu.async_copy(tabs_hbm.at[idx_v.at[b]], rows_v.at[slot], sem)

    def gather_wait(slot, sem):
        pltpu.make_async_copy(tabs_hbm.at[pl.ds(0, PAD)], rows_v.at[slot], sem).wait()

    lane = lax.iota(jnp.int32, 16)

    def compute(b, slot):
        buf = rows_v.at[slot]
        r0 = buf.at[0]
        c = [r0[pl.ds(k * 16, 16)] for k in range(DIM // 16)]
        ci = b & (CHUNK - 1)

        for grp in range(PAD // 16):
            acc = jnp.zeros((16,), jnp.float32)
            for jl in range(16):
                j = grp * 16 + jl
                if j >= PN:
                    break
                r = buf.at[j + 1]
                s = r[pl.ds(0, 16)] * c[0]
                for k in range(1, DIM // 16):
                    s = s + r[pl.ds(k * 16, 16)] * c[k]
                acc = jnp.where(lane == jl, jnp.sum(s), acc)
            scores_v[ci, pl.ds(grp * 16, 16)] = acc

        @pl.when(ci == CHUNK - 1)
        def _flush():
            off = pl.multiple_of(base + b - (CHUNK - 1), CHUNK)
            pltpu.sync_copy(scores_v, scores_hbm.at[pl.ds(off, CHUNK)])

    gather_start(0, 0, gsem0)

    def tbody(t, _):
        b0 = 2 * t
        b1 = 2 * t + 1
        gather_start(b1, 1, gsem1)
        gather_wait(0, gsem0)
        compute(b0, 0)

        @pl.when(t < BPW // 2 - 1)
        def _():
            gather_start(b1 + 1, 0, gsem0)

        gather_wait(1, gsem1)
        compute(b1, 1)
        return 0

    lax.fori_loop(0, BPW // 2, tbody, 0)


@jax.jit
def _sc_scores(idx, tabs):
    mesh = plsc.VectorSubcoreMesh(core_axis_name="c", subcore_axis_name="s")
    return pl.kernel(
        _sc_body,
        out_type=jax.ShapeDtypeStruct((B, PAD), jnp.float32),
        mesh=mesh,
        compiler_params=pltpu.CompilerParams(
            needs_layout_passes=False,
            use_tc_tiling_on_sc=False,
        ),
        scratch_types=[
            pltpu.VMEM((BPW, PAD), jnp.int32),
            pltpu.VMEM((2, PAD, DIM), jnp.float32),
            pltpu.VMEM((CHUNK, PAD), jnp.float32),
            pltpu.SemaphoreType.DMA,
            pltpu.SemaphoreType.DMA,
        ],
    )(idx, tabs)


def _tc_body(s_ref, o_ref):
    x = s_ref[...]
    col = lax.broadcasted_iota(jnp.int32, x.shape, 1)
    t = jnp.log1p(jnp.exp(-jnp.abs(x))) - jnp.minimum(x, 0.0)
    o_ref[...] = jnp.sum(jnp.where(col < PN, t, 0.0), axis=1)


@jax.jit
def _tc_reduce(scores):
    blk = 2048
    return pl.pallas_call(
        _tc_body,
        grid=(B // blk,),
        in_specs=[pl.BlockSpec((blk, PAD), lambda i: (i, 0))],
        out_specs=pl.BlockSpec((blk,), lambda i: (i,)),
        out_shape=jax.ShapeDtypeStruct((B,), jnp.float32),
    )(scores)


def kernel(center, pos_words, neg_words, in_table, out_table):
    tabs = jnp.concatenate(
        [in_table.T, out_table.T], axis=0).T.reshape(2 * VOCAB, DIM)
    idx = jnp.concatenate(
        [2 * center[:, None], 2 * pos_words + 1, 2 * neg_words + 1,
         2 * pos_words[:, : PAD - 1 - PN] + 1], axis=1)
    scores = _sc_scores(idx, tabs)
    return _tc_reduce(scores)
